# baseline (speedup 1.0000x reference)
import functools

import jax
import jax.numpy as jnp
from jax import lax
from jax.experimental import pallas as pl
from jax.experimental.pallas import tpu as pltpu
from jax.experimental.pallas import tpu_sc as plsc

_BATCH = 4
_SEQ = 4096
_DIM = 1024
_NC = 2
_NS = 16
_NW = _NC * _NS
_ROWS_PER_W = _SEQ // _NW


@functools.partial(
    pl.kernel,
    mesh=plsc.VectorSubcoreMesh(core_axis_name="c", subcore_axis_name="s"),
    out_type=jax.ShapeDtypeStruct((_BATCH, _SEQ, _DIM), jnp.float32),
    scratch_types=[pltpu.SemaphoreType.DMA],
)
def _broadcast_rows(w_hbm, out_hbm, sem):
    wid = lax.axis_index("s") * _NC + lax.axis_index("c")
    base = wid * _ROWS_PER_W
    src = w_hbm.at[pl.ds(base, _ROWS_PER_W)]
    copies = [
        pltpu.make_async_copy(src, out_hbm.at[b].at[pl.ds(base, _ROWS_PER_W)], sem)
        for b in range(_BATCH)
    ]
    for cp in copies:
        cp.start()
    for cp in copies:
        cp.wait()


def kernel(input, weights):
    del input
    return _broadcast_rows(weights)


# speedup vs baseline: 43.4622x; 43.4622x over previous
import functools

import jax
import jax.numpy as jnp
from jax import lax
from jax.experimental import pallas as pl
from jax.experimental.pallas import tpu as pltpu
from jax.experimental.pallas import tpu_sc as plsc

_BATCH = 4
_SEQ = 4096
_DIM = 1024
_NC = 2
_NS = 16
_NW = _NC * _NS
_ROWS_PER_W = _SEQ // _NW
_CHUNK = 32
_NCHUNK = _ROWS_PER_W // _CHUNK


@functools.partial(
    pl.kernel,
    mesh=plsc.VectorSubcoreMesh(core_axis_name="c", subcore_axis_name="s"),
    out_type=jax.ShapeDtypeStruct((_BATCH, _SEQ, _DIM), jnp.float32),
    scratch_types=[
        pltpu.VMEM((_CHUNK, _DIM), jnp.float32),
        pltpu.VMEM((_CHUNK, _DIM), jnp.float32),
        pltpu.SemaphoreType.DMA,
        pltpu.SemaphoreType.DMA,
    ],
)
def _broadcast_rows(w_hbm, out_hbm, buf_a, buf_b, wsem_a, wsem_b):
    wid = lax.axis_index("s") * _NC + lax.axis_index("c")
    base = wid * _ROWS_PER_W
    bufs = (buf_a, buf_b)
    wsems = (wsem_a, wsem_b)
    writes = []
    for i in range(_NCHUNK):
        buf = bufs[i % 2]
        wsem = wsems[i % 2]
        if i >= 2:
            for cp in writes[i - 2]:
                cp.wait()
        rows = pl.ds(base + i * _CHUNK, _CHUNK)
        pltpu.sync_copy(w_hbm.at[rows], buf)
        cps = [
            pltpu.make_async_copy(buf, out_hbm.at[b].at[rows], wsem)
            for b in range(_BATCH)
        ]
        for cp in cps:
            cp.start()
        writes.append(cps)
    for i in (_NCHUNK - 2, _NCHUNK - 1):
        for cp in writes[i]:
            cp.wait()


def kernel(input, weights):
    del input
    return _broadcast_rows(weights)


# speedup vs baseline: 45.7734x; 1.0532x over previous
import jax
import jax.numpy as jnp
from jax.experimental import pallas as pl

_BATCH = 4
_SEQ = 4096
_DIM = 1024
_SBLK = 1024


def _body(w_ref, o_ref):
    o_ref[...] = w_ref[...][None]


def kernel(input, weights):
    del input
    grid = (_SEQ // _SBLK, _BATCH)
    return pl.pallas_call(
        _body,
        grid=grid,
        in_specs=[pl.BlockSpec((_SBLK, _DIM), lambda i, b: (i, 0))],
        out_specs=pl.BlockSpec((1, _SBLK, _DIM), lambda i, b: (b, i, 0)),
        out_shape=jax.ShapeDtypeStruct((_BATCH, _SEQ, _DIM), jnp.float32),
    )(weights[:_SEQ])


# speedup vs baseline: 69.4162x; 1.5165x over previous
import jax
import jax.numpy as jnp
from jax.experimental import pallas as pl

_BATCH = 4
_SEQ = 4096
_DIM = 1024
_SBLK = 2048


def _body(w_ref, o_ref):
    o_ref[...] = w_ref[...][None]


def kernel(input, weights):
    del input
    grid = (_SEQ // _SBLK, _BATCH)
    return pl.pallas_call(
        _body,
        grid=grid,
        in_specs=[pl.BlockSpec((_SBLK, _DIM), lambda i, b: (i, 0))],
        out_specs=pl.BlockSpec((1, _SBLK, _DIM), lambda i, b: (b, i, 0)),
        out_shape=jax.ShapeDtypeStruct((_BATCH, _SEQ, _DIM), jnp.float32),
    )(weights)
